# Initial kernel scaffold; baseline (speedup 1.0000x reference)
import jax
import jax.numpy as jnp
from jax.experimental import pallas as pl


def kernel(inputs, img_feat0, img_feat1, img_feat2, img_feat3):
    raise NotImplementedError("write your pallas kernel here")


# speedup vs baseline: 3.2229x; 3.2229x over previous
import functools

import jax
import jax.numpy as jnp
from jax import lax
from jax.experimental import pallas as pl
from jax.experimental.pallas import tpu as pltpu
from jax.experimental.pallas import tpu_sc as plsc

_SIZES = (56, 28, 14, 7)
_CHANS = (64, 128, 256, 512)
_OFFS = (0, 64, 192, 448)
_B = 4
_N = 20000
_V = 16
_NW = 32
_NBLK = _N // _V


def _prep(coord):
    X, Y, Z = coord[:, 0], coord[:, 1], coord[:, 2]
    h = 250.0 * (-Y / -Z) + 112.0
    w = 250.0 * (X / -Z) + 112.0
    h = jnp.clip(h, 0.0, 223.0)
    w = jnp.clip(w, 0.0, 223.0)
    idx_rows, wt_rows = [], []
    for S in _SIZES:
        x = jnp.clip(h / (224.0 / S), 0.0, S - 1.0)
        y = jnp.clip(w / (224.0 / S), 0.0, S - 1.0)
        x1 = jnp.floor(x)
        x2 = jnp.ceil(x)
        y1 = jnp.floor(y)
        y2 = jnp.ceil(y)
        i1 = x1.astype(jnp.int32)
        i2 = x2.astype(jnp.int32)
        j1 = y1.astype(jnp.int32)
        j2 = y2.astype(jnp.int32)
        lvl_idx = jnp.stack([i1 * S + j1, i1 * S + j2, i2 * S + j1,
                             i2 * S + j2])
        idx_rows.append(lvl_idx.reshape(4, _NBLK, _V).transpose(1, 0, 2)
                        .reshape(_NBLK * 4 * _V))
        wt_rows += [(x2 - x) * (y2 - y), (x2 - x) * (y - y1),
                    (x - x1) * (y2 - y), (x - x1) * (y - y1)]
    wts = jnp.stack(wt_rows)
    wsplat = jnp.broadcast_to(wts.T[:, :, None], (_N, 16, 16))
    return idx_rows, wsplat.reshape(_N * 256)


def _make_sc_kernel():
    mesh = plsc.VectorSubcoreMesh(core_axis_name="c", subcore_axis_name="s")
    max_iter = (_NBLK + _NW - 1) // _NW

    scratch = [
        pltpu.VMEM((4 * _V,), jnp.int32),
        pltpu.VMEM((4 * _V,), jnp.int32),
        pltpu.VMEM((4 * _V,), jnp.int32),
        pltpu.VMEM((4 * _V,), jnp.int32),
        pltpu.VMEM((256 * _V,), jnp.float32),
        pltpu.VMEM((4 * _V, 64), jnp.float32),
        pltpu.VMEM((4 * _V, 128), jnp.float32),
        pltpu.VMEM((4 * _V, 256), jnp.float32),
        pltpu.VMEM((4 * _V, 512), jnp.float32),
        pltpu.VMEM((_V, 960), jnp.float32),
        pltpu.SemaphoreType.DMA,
    ]

    @functools.partial(
        pl.kernel, mesh=mesh,
        out_type=jax.ShapeDtypeStruct((_B, _N, 960), jnp.float32),
        scratch_types=scratch,
        compiler_params=pltpu.CompilerParams(use_tc_tiling_on_sc=False))
    def k(idx0_hbm, idx1_hbm, idx2_hbm, idx3_hbm, wts_hbm, *rest):
        tabs = rest[:16]
        out_hbm = rest[16]
        i0, i1, i2, i3, wts_v, g0, g1, g2, g3, ob, sem = rest[17:]
        ibufs = (i0, i1, i2, i3)
        gbufs = (g0, g1, g2, g3)
        idx_hbms = (idx0_hbm, idx1_hbm, idx2_hbm, idx3_hbm)

        wid = lax.axis_index("s") * 2 + lax.axis_index("c")

        def blk_body(it, carry):
            blk = it * _NW + wid

            @pl.when(blk < _NBLK)
            def _():
                n0 = blk * _V
                for l in range(4):
                    pltpu.sync_copy(
                        idx_hbms[l].at[pl.ds(blk * 4 * _V, 4 * _V)], ibufs[l])
                pltpu.sync_copy(
                    wts_hbm.at[pl.ds(n0 * 256, 256 * _V)], wts_v)

                for b in range(_B):
                    for l in range(4):
                        pltpu.async_copy(
                            tabs[l * _B + b].at[ibufs[l]], gbufs[l], sem
                        ).wait()

                    def v_body(v, c2):
                        wb = v * 256
                        for l, C in enumerate(_CHANS):
                            g = gbufs[l]
                            w0 = wts_v[pl.ds(wb + (4 * l + 0) * 16, 16)]
                            w1 = wts_v[pl.ds(wb + (4 * l + 1) * 16, 16)]
                            w2 = wts_v[pl.ds(wb + (4 * l + 2) * 16, 16)]
                            w3 = wts_v[pl.ds(wb + (4 * l + 3) * 16, 16)]
                            for c in range(C // 16):
                                sl = pl.ds(c * 16, 16)
                                acc = w0 * g[0 * _V + v, sl]
                                acc = acc + w2 * g[2 * _V + v, sl]
                                acc = acc + w1 * g[1 * _V + v, sl]
                                acc = acc + w3 * g[3 * _V + v, sl]
                                ob[v, pl.ds(_OFFS[l] + c * 16, 16)] = acc
                        return c2

                    lax.fori_loop(0, _V, v_body, 0)

                    pltpu.sync_copy(ob, out_hbm.at[b, pl.ds(n0, _V), :])

            return carry

        lax.fori_loop(0, max_iter, blk_body, 0)

    return k


_SC_KERNEL = _make_sc_kernel()


def kernel(inputs, img_feat0, img_feat1, img_feat2, img_feat3):
    coord = inputs[0]
    idx_list, wts = _prep(coord)
    feats = (img_feat0, img_feat1, img_feat2, img_feat3)
    tables = []
    for l, (S, C) in enumerate(zip(_SIZES, _CHANS)):
        t = jnp.transpose(feats[l], (0, 2, 3, 1)).reshape(_B, S * S, C)
        for b in range(_B):
            tables.append(t[b])
    feat_out = _SC_KERNEL(*idx_list, wts, *tables)
    coord_b = jnp.broadcast_to(coord[None], (_B, _N, 3))
    return jnp.concatenate([coord_b, feat_out], axis=2)


# speedup vs baseline: 4.0025x; 1.2419x over previous
import functools

import jax
import jax.numpy as jnp
from jax import lax
from jax.experimental import pallas as pl
from jax.experimental.pallas import tpu as pltpu
from jax.experimental.pallas import tpu_sc as plsc

_SIZES = (56, 28, 14, 7)
_CHANS = (64, 128, 256, 512)
_OFFS = (0, 64, 192, 448)
_B = 4
_N = 20000
_V = 16
_NW = 32
_NBLK = _N // _V


def _prep(coord):
    X, Y, Z = coord[:, 0], coord[:, 1], coord[:, 2]
    h = 250.0 * (-Y / -Z) + 112.0
    w = 250.0 * (X / -Z) + 112.0
    h = jnp.clip(h, 0.0, 223.0)
    w = jnp.clip(w, 0.0, 223.0)
    idx_rows, wt_rows = [], []
    for S in _SIZES:
        x = jnp.clip(h / (224.0 / S), 0.0, S - 1.0)
        y = jnp.clip(w / (224.0 / S), 0.0, S - 1.0)
        x1 = jnp.floor(x)
        x2 = jnp.ceil(x)
        y1 = jnp.floor(y)
        y2 = jnp.ceil(y)
        i1 = x1.astype(jnp.int32)
        i2 = x2.astype(jnp.int32)
        j1 = y1.astype(jnp.int32)
        j2 = y2.astype(jnp.int32)
        lvl_idx = jnp.stack([i1 * S + j1, i1 * S + j2, i2 * S + j1,
                             i2 * S + j2])
        idx_rows.append(lvl_idx.reshape(4, _NBLK, _V).transpose(1, 0, 2)
                        .reshape(_NBLK * 4 * _V))
        wt_rows += [(x2 - x) * (y2 - y), (x2 - x) * (y - y1),
                    (x - x1) * (y2 - y), (x - x1) * (y - y1)]
    wts = jnp.stack(wt_rows)
    wsplat = jnp.broadcast_to(wts.T[:, :, None], (_N, 16, 16))
    return idx_rows, wsplat.reshape(_N * 256)


def _make_sc_kernel():
    mesh = plsc.VectorSubcoreMesh(core_axis_name="c", subcore_axis_name="s")
    max_iter = (_NBLK + _NW - 1) // _NW

    scratch = [
        pltpu.VMEM((4 * _V,), jnp.int32),
        pltpu.VMEM((4 * _V,), jnp.int32),
        pltpu.VMEM((4 * _V,), jnp.int32),
        pltpu.VMEM((4 * _V,), jnp.int32),
        pltpu.VMEM((256 * _V,), jnp.float32),
        pltpu.VMEM((4 * _V, 64), jnp.float32),
        pltpu.VMEM((4 * _V, 128), jnp.float32),
        pltpu.VMEM((4 * _V, 256), jnp.float32),
        pltpu.VMEM((4 * _V, 512), jnp.float32),
        pltpu.VMEM((_V, 960), jnp.float32),
        pltpu.SemaphoreType.DMA,
    ]

    @functools.partial(
        pl.kernel, mesh=mesh,
        out_type=jax.ShapeDtypeStruct((_B, _N, 960), jnp.float32),
        scratch_types=scratch,
        compiler_params=pltpu.CompilerParams(use_tc_tiling_on_sc=False))
    def k(idx0_hbm, idx1_hbm, idx2_hbm, idx3_hbm, wts_hbm, *rest):
        tabs = rest[:16]
        out_hbm = rest[16]
        i0, i1, i2, i3, wts_v, g0, g1, g2, g3, ob, sem = rest[17:]
        ibufs = (i0, i1, i2, i3)
        gbufs = (g0, g1, g2, g3)
        idx_hbms = (idx0_hbm, idx1_hbm, idx2_hbm, idx3_hbm)

        wid = lax.axis_index("s") * 2 + lax.axis_index("c")

        def blk_body(it, carry):
            blk = it * _NW + wid

            @pl.when(blk < _NBLK)
            def _():
                n0 = blk * _V
                for l in range(4):
                    pltpu.sync_copy(
                        idx_hbms[l].at[pl.ds(blk * 4 * _V, 4 * _V)], ibufs[l])
                pltpu.sync_copy(
                    wts_hbm.at[pl.ds(n0 * 256, 256 * _V)], wts_v)

                for b in range(_B):
                    for l in range(4):
                        pltpu.async_copy(
                            tabs[l * _B + b].at[ibufs[l]], gbufs[l], sem
                        ).wait()

                    def v_body(v, c2):
                        wb = v * 256
                        for l, C in enumerate(_CHANS):
                            g = gbufs[l]
                            w0 = wts_v[pl.ds(wb + (4 * l + 0) * 16, 16)]
                            w1 = wts_v[pl.ds(wb + (4 * l + 1) * 16, 16)]
                            w2 = wts_v[pl.ds(wb + (4 * l + 2) * 16, 16)]
                            w3 = wts_v[pl.ds(wb + (4 * l + 3) * 16, 16)]
                            for c in range(C // 16):
                                sl = pl.ds(c * 16, 16)
                                acc = w0 * g[0 * _V + v, sl]
                                acc = acc + w2 * g[2 * _V + v, sl]
                                acc = acc + w1 * g[1 * _V + v, sl]
                                acc = acc + w3 * g[3 * _V + v, sl]
                                ob[v, pl.ds(_OFFS[l] + c * 16, 16)] = acc
                        return c2

                    lax.fori_loop(0, _V, v_body, 0)

                    pltpu.sync_copy(ob, out_hbm.at[b, pl.ds(n0, _V), :])

            return carry

        lax.fori_loop(0, max_iter, blk_body, 0)

    return k


_SC_KERNEL = _make_sc_kernel()

_T = 400


def _concat_body(coord_ref, feat_ref, out_ref):
    out_ref[:, :, 0:3] = coord_ref[...][None]
    out_ref[:, :, 3:963] = feat_ref[...]


def _tc_concat(coord, feat):
    return pl.pallas_call(
        _concat_body,
        grid=(_B, _N // _T),
        in_specs=[
            pl.BlockSpec((_T, 3), lambda b, i: (i, 0)),
            pl.BlockSpec((1, _T, 960), lambda b, i: (b, i, 0)),
        ],
        out_specs=pl.BlockSpec((1, _T, 963), lambda b, i: (b, i, 0)),
        out_shape=jax.ShapeDtypeStruct((_B, _N, 963), jnp.float32),
    )(coord, feat)


def kernel(inputs, img_feat0, img_feat1, img_feat2, img_feat3):
    coord = inputs[0]
    idx_list, wts = _prep(coord)
    feats = (img_feat0, img_feat1, img_feat2, img_feat3)
    tables = []
    for l, (S, C) in enumerate(zip(_SIZES, _CHANS)):
        t = jnp.transpose(feats[l], (0, 2, 3, 1)).reshape(_B, S * S, C)
        for b in range(_B):
            tables.append(t[b])
    feat_out = _SC_KERNEL(*idx_list, wts, *tables)
    return _tc_concat(coord, feat_out)
